# Initial kernel scaffold; baseline (speedup 1.0000x reference)
import jax
import jax.numpy as jnp
from jax.experimental import pallas as pl


def kernel(node_inp, edge_index, edge_type, W_self, W_cross, W_word, att_src_word, att_dst_word, bias_word, att_l_lang, att_r_lang, bias_lang):
    raise NotImplementedError("write your pallas kernel here")


# speedup vs baseline: 19.8989x; 19.8989x over previous
import functools

import jax
import jax.numpy as jnp
from jax import lax
from jax.experimental import pallas as pl
from jax.experimental.pallas import tpu as pltpu
from jax.experimental.pallas import tpu_sc as plsc

N = 10000
E = 320000
IN_DIM = 128
OUT_DIM = 128
HEADS = 8
HD = 16
NR = 4
SEG = NR * N
HALF = SEG // 2
PADH = HALF + 224
SLAB = PADH // 16
EPT = E // 16
CHUNK = 2000
BATCH = 80
NCH = EPT // CHUNK
NB = CHUNK // BATCH
NG = BATCH // 16


def _pre_body(node_ref, ww_ref, ms_ref, md_ref, xw_ref, as_ref, ad_ref):
    x = node_ref[...]
    xw = jnp.dot(x, ww_ref[0], preferred_element_type=jnp.float32)
    xw_ref[0] = xw
    as_ref[0] = jnp.dot(xw, ms_ref[0], preferred_element_type=jnp.float32)
    ad_ref[0] = jnp.dot(xw, md_ref[0], preferred_element_type=jnp.float32)


def _seg_body(src_ref, dst_ref, et_ref, ss_ref, sd_ref):
    et = et_ref[...]
    ss_ref[...] = et * N + src_ref[...]
    sd_ref[...] = et * N + dst_ref[...]


def _post_body(node_ref, wself_ref, outw_ref, bw_ref, wc_ref, s0_ref,
               ml_ref, mr_ref, bm_ref, bl_ref, o_ref):
    x0 = jnp.dot(node_ref[...], wself_ref[...],
                 preferred_element_type=jnp.float32)
    ad = jnp.dot(x0, mr_ref[...], preferred_element_type=jnp.float32)
    s0 = s0_ref[...]
    xs = [x0]
    a0 = jnp.dot(x0, ml_ref[...], preferred_element_type=jnp.float32)
    l0 = a0 + ad
    lgs = [jnp.where(l0 >= 0.0, l0, 0.2 * l0)]
    for r in range(NR):
        t = outw_ref[r] + bw_ref[r]
        g = 0.5 * t * (1.0 + lax.erf(t * 0.7071067811865476))
        cr = jnp.dot(g, wc_ref[...], preferred_element_type=jnp.float32)
        xs.append(cr)
        ak = jnp.dot(cr, ml_ref[...], preferred_element_type=jnp.float32)
        lk = ak + ad
        lk = jnp.where(lk >= 0.0, lk, 0.2 * lk)
        v = s0[:, r:r + 1] > 0.0
        lgs.append(jnp.where(v, lk, -jnp.inf))
    m = lgs[0]
    for l in lgs[1:]:
        m = jnp.maximum(m, l)
    exs = [jnp.exp(lgs[0] - m)]
    for k in range(1, NR + 1):
        lk = lgs[k]
        exs.append(jnp.where(jnp.isfinite(lk), jnp.exp(lk - m), 0.0))
    ssum = exs[0]
    for e in exs[1:]:
        ssum = ssum + e
    inv = 1.0 / (ssum + 1e-16)
    acc = None
    for xk, e in zip(xs, exs):
        alb = jnp.dot(e * inv, bm_ref[...], preferred_element_type=jnp.float32)
        acc = xk * alb if acc is None else acc + xk * alb
    o_ref[...] = acc + bl_ref[0]


def _sc_body(ssrc_hbm, sdst_hbm, asrc_hbm, adst_hbm, xw_hbm, s_hbm,
             o0, o1, o2, o3, o4, o5, o6, o7,
             asrc_v, adst_v, s_tab, ssrc_c, sdst_c, exb, alph,
             sidx, xidx, rows_v, msgf, sidxf, sh_s, sh_out, sem):
    out_refs = (o0, o1, o2, o3, o4, o5, o6, o7)
    c = lax.axis_index("c")
    s = lax.axis_index("s")
    lo = c * HALF
    ebase = s * EPT
    zv = jnp.zeros((16,), jnp.float32)
    col16 = lax.iota(jnp.int32, 16)

    def ztab(i, carry):
        ri = (jnp.full((16,), 0, jnp.int32) + i) * 16 + col16
        plsc.store_scatter(s_tab, [ri], zv)
        return carry

    for h in range(HEADS):
        lax.fori_loop(0, PADH // 16, ztab, 0)
        pltpu.sync_copy(asrc_hbm.at[pl.ds(h * SEG, SEG)], asrc_v)
        pltpu.sync_copy(adst_hbm.at[pl.ds(h * SEG + c * HALF, HALF)],
                        adst_v.at[pl.ds(0, HALF)])
        pltpu.sync_copy(s_tab.at[pl.ds(0, SLAB)],
                        sh_s.at[pl.ds(s * SLAB, SLAB)])
        pltpu.sync_copy(s_tab, sh_out.at[pl.ds(s * PADH, PADH)])
        plsc.subcore_barrier()

        def chunk_a(ci, carry):
            pltpu.sync_copy(ssrc_hbm.at[pl.ds(ebase + ci * CHUNK, CHUNK)],
                            ssrc_c)
            pltpu.sync_copy(sdst_hbm.at[pl.ds(ebase + ci * CHUNK, CHUNK)],
                            sdst_c)

            def batch_a(bi, carry2):
                for g in range(NG):
                    offv = ((jnp.full((16,), 0, jnp.int32) + bi) * BATCH
                            + g * 16 + col16)
                    vs = plsc.load_gather(ssrc_c, [offv])
                    vd = plsc.load_gather(sdst_c, [offv])
                    ldx = vd - lo
                    own = (ldx >= 0) & (ldx < HALF)
                    ldxc = jnp.where(own, ldx, HALF + col16)
                    ga = plsc.load_gather(asrc_v, [vs])
                    gd = plsc.load_gather(adst_v, [ldxc])
                    x = ga + gd
                    ex = jnp.exp(jnp.where(x >= 0.0, x, 0.2 * x))
                    exb[pl.ds(g * 16, 16)] = jnp.where(own, ex, 0.0)
                    sidx[pl.ds(g * 16, 16)] = ldxc
                pltpu.sync_copy(exb, sh_s.at[sidx], add=True)
                return carry2

            lax.fori_loop(0, NB, batch_a, 0)
            return carry

        lax.fori_loop(0, NCH, chunk_a, 0)
        plsc.subcore_barrier()
        pltpu.sync_copy(sh_s, s_tab)
        if h == 0:
            pltpu.sync_copy(s_tab.at[pl.ds(s * SLAB, SLAB)],
                            s_hbm.at[pl.ds(c * PADH + s * SLAB, SLAB)])

        def chunk_b(ci, carry):
            pltpu.sync_copy(ssrc_hbm.at[pl.ds(ebase + ci * CHUNK, CHUNK)],
                            ssrc_c)
            pltpu.sync_copy(sdst_hbm.at[pl.ds(ebase + ci * CHUNK, CHUNK)],
                            sdst_c)

            def batch_b(bi, carry2):
                for g in range(NG):
                    offv = ((jnp.full((16,), 0, jnp.int32) + bi) * BATCH
                            + g * 16 + col16)
                    vs = plsc.load_gather(ssrc_c, [offv])
                    vd = plsc.load_gather(sdst_c, [offv])
                    ldx = vd - lo
                    own = (ldx >= 0) & (ldx < HALF)
                    ldxc = jnp.where(own, ldx, HALF + col16)
                    ga = plsc.load_gather(asrc_v, [vs])
                    gd = plsc.load_gather(adst_v, [ldxc])
                    x = ga + gd
                    ex = jnp.exp(jnp.where(x >= 0.0, x, 0.2 * x))
                    sg = plsc.load_gather(s_tab, [ldxc])
                    al = ex / (sg + 1e-16)
                    alph[pl.ds(g * 16, 16)] = jnp.where(own, al, 0.0)
                    sidx[pl.ds(g * 16, 16)] = ldxc
                    xidx[pl.ds(g * 16, 16)] = vs
                pltpu.async_copy(xw_hbm.at[xidx], rows_v, sem).wait()
                for g in range(NG):
                    alg = alph[pl.ds(g * 16, 16)]
                    ldg = sidx[pl.ds(g * 16, 16)]
                    base = ldg * 16
                    ridx = g * 16 + col16
                    for j in range(16):
                        cj = plsc.load_gather(rows_v, [ridx, h * HD + j
                                                       + 0 * col16])
                        p = g * 256 + col16 * 16 + j
                        plsc.store_scatter(msgf, [p >> 7, p & 127], cj * alg)
                        plsc.store_scatter(sidxf, [p >> 7, p & 127], base + j)
                for q in range(BATCH * 16 // 128):
                    pltpu.sync_copy(msgf.at[q], sh_out.at[sidxf.at[q]],
                                    add=True)
                return carry2

            lax.fori_loop(0, NB, batch_b, 0)
            return carry

        lax.fori_loop(0, NCH, chunk_b, 0)
        plsc.subcore_barrier()
        pltpu.sync_copy(sh_out.at[pl.ds(s * PADH, PADH)], s_tab)
        pltpu.sync_copy(s_tab,
                        out_refs[h].at[pl.ds(c * PADH * 16 + s * PADH, PADH)])
        plsc.subcore_barrier()


def _banded(att):
    m = jnp.zeros((OUT_DIM, OUT_DIM), jnp.float32)
    for h in range(HEADS):
        m = m.at[h * HD:(h + 1) * HD, h].set(att[h])
    return m


def kernel(node_inp, edge_index, edge_type, W_self, W_cross, W_word,
           att_src_word, att_dst_word, bias_word, att_l_lang, att_r_lang,
           bias_lang):
    f32 = jnp.float32
    ms = jnp.stack([_banded(att_src_word[r]) for r in range(NR)])
    md = jnp.stack([_banded(att_dst_word[r]) for r in range(NR)])
    ml = _banded(att_l_lang)
    mr = _banded(att_r_lang)
    bm = jnp.zeros((OUT_DIM, OUT_DIM), f32)
    for h in range(HEADS):
        bm = bm.at[h, h * HD:(h + 1) * HD].set(1.0)

    xw_pad, asrc_pad, adst_pad = pl.pallas_call(
        _pre_body,
        grid=(NR, 10),
        in_specs=[
            pl.BlockSpec((N // 10, IN_DIM), lambda r, b: (b, 0)),
            pl.BlockSpec((1, IN_DIM, OUT_DIM), lambda r, b: (r, 0, 0)),
            pl.BlockSpec((1, OUT_DIM, OUT_DIM), lambda r, b: (r, 0, 0)),
            pl.BlockSpec((1, OUT_DIM, OUT_DIM), lambda r, b: (r, 0, 0)),
        ],
        out_specs=[
            pl.BlockSpec((1, N // 10, OUT_DIM), lambda r, b: (r, b, 0)),
            pl.BlockSpec((1, N // 10, OUT_DIM), lambda r, b: (r, b, 0)),
            pl.BlockSpec((1, N // 10, OUT_DIM), lambda r, b: (r, b, 0)),
        ],
        out_shape=[
            jax.ShapeDtypeStruct((NR, N, OUT_DIM), f32),
            jax.ShapeDtypeStruct((NR, N, OUT_DIM), f32),
            jax.ShapeDtypeStruct((NR, N, OUT_DIM), f32),
        ],
    )(node_inp, W_word, ms, md)

    src2d = edge_index[0].reshape(E // 128, 128)
    dst2d = edge_index[1].reshape(E // 128, 128)
    et2d = edge_type.reshape(E // 128, 128)
    seg_src2d, seg_dst2d = pl.pallas_call(
        _seg_body,
        out_shape=[
            jax.ShapeDtypeStruct((E // 128, 128), jnp.int32),
            jax.ShapeDtypeStruct((E // 128, 128), jnp.int32),
        ],
    )(src2d, dst2d, et2d)
    seg_src = seg_src2d.reshape(E)
    seg_dst = seg_dst2d.reshape(E)

    asrc_t = asrc_pad[..., :HEADS].transpose(2, 0, 1).reshape(HEADS * SEG)
    adst_t = adst_pad[..., :HEADS].transpose(2, 0, 1).reshape(HEADS * SEG)
    xw_t = xw_pad.reshape(SEG, OUT_DIM)

    mesh = plsc.VectorSubcoreMesh(core_axis_name="c", subcore_axis_name="s")
    sc_run = functools.partial(
        pl.kernel,
        mesh=mesh,
        compiler_params=pltpu.CompilerParams(needs_layout_passes=False),
        out_type=(
            (jax.ShapeDtypeStruct((2 * PADH,), f32),)
            + tuple(jax.ShapeDtypeStruct((2 * PADH * 16,), f32)
                    for _ in range(HEADS))
        ),
        scratch_types=[
            pltpu.VMEM((SEG,), f32),
            pltpu.VMEM((HALF + 16,), f32),
            pltpu.VMEM((PADH,), f32),
            pltpu.VMEM((CHUNK,), jnp.int32),
            pltpu.VMEM((CHUNK,), jnp.int32),
            pltpu.VMEM((BATCH,), f32),
            pltpu.VMEM((BATCH,), f32),
            pltpu.VMEM((BATCH,), jnp.int32),
            pltpu.VMEM((BATCH,), jnp.int32),
            pltpu.VMEM((BATCH, OUT_DIM), f32),
            pltpu.VMEM((BATCH * 16 // 128, 128), f32),
            pltpu.VMEM((BATCH * 16 // 128, 128), jnp.int32),
            pltpu.VMEM_SHARED((PADH,), f32),
            pltpu.VMEM_SHARED((PADH * 16,), f32),
            pltpu.SemaphoreType.DMA,
        ],
    )(_sc_body)
    s_hbm, *out_heads = sc_run(seg_src, seg_dst, asrc_t, adst_t, xw_t)

    s0_rows = jnp.concatenate([s_hbm[:HALF], s_hbm[PADH:PADH + HALF]])
    s0p = s0_rows.reshape(NR, N).transpose(1, 0)
    s0p = jnp.pad(s0p, ((0, 0), (0, OUT_DIM - NR)))
    o = jnp.stack([
        jnp.concatenate([oh.reshape(2, PADH, 16)[0, :HALF],
                         oh.reshape(2, PADH, 16)[1, :HALF]])
        for oh in out_heads])
    outw = o.reshape(HEADS, NR, N, HD).transpose(1, 2, 0, 3)
    outw = outw.reshape(NR, N, OUT_DIM)

    out = pl.pallas_call(
        _post_body,
        grid=(25,),
        in_specs=[
            pl.BlockSpec((N // 25, IN_DIM), lambda b: (b, 0)),
            pl.BlockSpec((IN_DIM, OUT_DIM), lambda b: (0, 0)),
            pl.BlockSpec((NR, N // 25, OUT_DIM), lambda b: (0, b, 0)),
            pl.BlockSpec((NR, OUT_DIM), lambda b: (0, 0)),
            pl.BlockSpec((OUT_DIM, OUT_DIM), lambda b: (0, 0)),
            pl.BlockSpec((N // 25, OUT_DIM), lambda b: (b, 0)),
            pl.BlockSpec((OUT_DIM, OUT_DIM), lambda b: (0, 0)),
            pl.BlockSpec((OUT_DIM, OUT_DIM), lambda b: (0, 0)),
            pl.BlockSpec((OUT_DIM, OUT_DIM), lambda b: (0, 0)),
            pl.BlockSpec((1, OUT_DIM), lambda b: (0, 0)),
        ],
        out_specs=pl.BlockSpec((N // 25, OUT_DIM), lambda b: (b, 0)),
        out_shape=jax.ShapeDtypeStruct((N, OUT_DIM), f32),
    )(node_inp, W_self, outw, bias_word, W_cross, s0p, ml, mr, bm,
      bias_lang.reshape(1, OUT_DIM))
    return out
